# baseline (speedup 1.0000x reference)
import dataclasses
import functools

import jax
import jax.numpy as jnp
from jax import lax
from jax.experimental import pallas as pl
from jax.experimental.pallas import tpu as pltpu
from jax.experimental.pallas import tpu_sc as plsc

N = 10000
E = 320000
D = 128
NC = 2
NS = 16
NW = NC * NS
CHUNK = 80
EPT = E // NW
CPT = EPT // CHUNK
ACH = 100
ACP = EPT // ACH
APR = ACP // 2
RPT = N // NS

_vmesh = plsc.VectorSubcoreMesh(core_axis_name="c", subcore_axis_name="s")

_sc_cp = pltpu.CompilerParams()
if "needs_layout_passes" in pltpu.CompilerParams.__dataclass_fields__:
    _sc_cp = dataclasses.replace(_sc_cp, needs_layout_passes=False)


@functools.partial(
    pl.kernel,
    out_type=(jax.ShapeDtypeStruct((NW, 1, N), jnp.float32),
              jax.ShapeDtypeStruct((NW, 1, N), jnp.float32)),
    mesh=_vmesh,
    scratch_types=[
        pltpu.VMEM((1, N), jnp.float32),
        pltpu.VMEM((1, N), jnp.float32),
        pltpu.VMEM((CPT, CHUNK), jnp.int32),
    ],
    compiler_params=_sc_cp,
)
def _sc_degree(src_hbm, dst_hbm, osrc_hbm, odst_hbm, cs_v, cd_v, idx_v):
    c = lax.axis_index("c")
    s = lax.axis_index("s")
    w = c * NS + s
    zero16 = jnp.zeros((16,), jnp.float32)

    @pl.loop(0, N // 16)
    def _(g):
        cs_v[0, pl.ds(g * 16, 16)] = zero16
        cd_v[0, pl.ds(g * 16, 16)] = zero16

    ones = jnp.ones((16,), jnp.float32)
    zidx = jnp.zeros((16,), jnp.int32)

    pltpu.sync_copy(src_hbm.at[w], idx_v)

    @pl.loop(0, CPT)
    def _(r):
        for g in range(CHUNK // 16):
            plsc.addupdate_scatter(
                cs_v, [zidx, idx_v[r, pl.ds(g * 16, 16)]], ones)

    pltpu.sync_copy(dst_hbm.at[w], idx_v)

    @pl.loop(0, CPT)
    def _(r):
        for g in range(CHUNK // 16):
            plsc.addupdate_scatter(
                cd_v, [zidx, idx_v[r, pl.ds(g * 16, 16)]], ones)

    pltpu.sync_copy(cs_v, osrc_hbm.at[w])
    pltpu.sync_copy(cd_v, odst_hbm.at[w])


@functools.partial(
    pl.kernel,
    out_type=jax.ShapeDtypeStruct((NC, N, D), jnp.float32),
    mesh=_vmesh,
    scratch_types=[
        [pltpu.VMEM((ACH, D), jnp.float32) for _ in range(2)],
        pltpu.VMEM((ACP, ACH), jnp.int32),
        [pltpu.VMEM((2, ACH), jnp.int32) for _ in range(2)],
        [pltpu.VMEM((2, ACH), jnp.float32) for _ in range(2)],
        pltpu.VMEM_SHARED((N, D), jnp.float32),
        [pltpu.SemaphoreType.DMA for _ in range(2)],
        [pltpu.SemaphoreType.DMA for _ in range(2)],
        [pltpu.SemaphoreType.DMA for _ in range(2)],
    ],
    compiler_params=_sc_cp,
)
def _sc_aggregate(feat_hbm, src_hbm, dst_hbm, ew_hbm, out_hbm,
                  rows, src_v, dstp, ewp, acc_sh, sg, ss, sa):
    c = lax.axis_index("c")
    s = lax.axis_index("s")
    w = c * NS + s

    zero16 = jnp.zeros((16,), jnp.float32)

    @pl.loop(0, ACH)
    def _(r):
        for v in range(D // 16):
            rows[0][r, pl.ds(v * 16, 16)] = zero16

    @pl.loop(s, N // 80, step=NS)
    def _(k):
        pltpu.async_copy(rows[0].at[pl.ds(0, 80)],
                         acc_sh.at[pl.ds(k * 80, 80)], sg[0])

    @pl.loop(s, N // 80, step=NS)
    def _(k):
        pltpu.make_async_copy(rows[0].at[pl.ds(0, 80)],
                              acc_sh.at[pl.ds(k * 80, 80)], sg[0]).wait()

    plsc.subcore_barrier()

    pltpu.sync_copy(src_hbm.at[w], src_v)

    def _aux_start(p, a):
        pltpu.async_copy(dst_hbm.at[w].at[p], dstp[a], sa[a])
        pltpu.async_copy(ew_hbm.at[w].at[p], ewp[a], sa[a])

    def _aux_wait(a):
        pltpu.make_async_copy(dst_hbm.at[w].at[0], dstp[a], sa[a]).wait()
        pltpu.make_async_copy(ew_hbm.at[w].at[0], ewp[a], sa[a]).wait()

    def _gather_start(k, j):
        pltpu.async_copy(feat_hbm.at[src_v.at[k]], rows[j], sg[j])

    def _gather_wait(j):
        pltpu.make_async_copy(feat_hbm.at[src_v.at[0]], rows[j],
                              sg[j]).wait()

    def _scatter_start(j, a, i):
        pltpu.async_copy(rows[j], acc_sh.at[dstp[a].at[i]], ss[j], add=True)

    def _scatter_wait(j, a, i):
        pltpu.make_async_copy(rows[j], acc_sh.at[dstp[a].at[i]],
                              ss[j]).wait()

    def _scale(j, a, i):
        ii = jnp.broadcast_to(i, (16,)).astype(jnp.int32)

        @pl.loop(0, ACH, unroll=4)
        def _(r):
            ewb = plsc.load_gather(
                ewp[a], [ii, jnp.broadcast_to(r, (16,)).astype(jnp.int32)])
            for v in range(D // 16):
                sl = pl.ds(v * 16, 16)
                rows[j][r, sl] = rows[j][r, sl] * ewb

    _aux_start(0, 0)
    _aux_start(1, 1)
    _gather_start(0, 0)
    _gather_start(1, 1)

    @pl.loop(0, ACP // 4)
    def _(u):
        k0 = 4 * u
        p0 = 2 * u
        _aux_wait(0)
        _gather_wait(0)
        _scale(0, 0, 0)
        _scatter_start(0, 0, 0)
        _gather_wait(1)
        _scale(1, 0, 1)
        _scatter_start(1, 0, 1)
        _scatter_wait(0, 0, 0)
        _gather_start(k0 + 2, 0)
        _scatter_wait(1, 0, 1)
        _gather_start(k0 + 3, 1)
        _aux_start(jnp.minimum(p0 + 2, APR - 2), 0)
        _aux_wait(1)
        _gather_wait(0)
        _scale(0, 1, 0)
        _scatter_start(0, 1, 0)
        _gather_wait(1)
        _scale(1, 1, 1)
        _scatter_start(1, 1, 1)
        _scatter_wait(0, 1, 0)
        _gather_start(jnp.minimum(k0 + 4, ACP - 4), 0)
        _scatter_wait(1, 1, 1)
        _gather_start(jnp.minimum(k0 + 5, ACP - 3), 1)
        _aux_start(jnp.minimum(p0 + 3, APR - 1), 1)

    _gather_wait(0)
    _gather_wait(1)
    _aux_wait(0)
    _aux_wait(1)

    plsc.subcore_barrier()

    @pl.loop(s, N // 80, step=NS)
    def _(k):
        pltpu.async_copy(acc_sh.at[pl.ds(k * 80, 80)],
                         out_hbm.at[c].at[pl.ds(k * 80, 80)], sg[0])

    @pl.loop(s, N // 80, step=NS)
    def _(k):
        pltpu.make_async_copy(acc_sh.at[pl.ds(k * 80, 80)],
                              out_hbm.at[c].at[pl.ds(k * 80, 80)],
                              sg[0]).wait()


def _tc_prep_body(cs_ref, cd_ref, x_ref, wself_ref, feat_ref, hs_ref, nin_ref):
    deg_s = jnp.maximum(jnp.sum(cs_ref[...], axis=0), 1.0)
    deg_d = jnp.maximum(jnp.sum(cd_ref[...], axis=0), 1.0)
    x = x_ref[...]
    feat_ref[...] = x * lax.rsqrt(deg_s)[:, None]
    hs_ref[...] = lax.dot_general(
        x, wself_ref[...], (((1,), (1,)), ((), ())),
        preferred_element_type=jnp.float32,
        precision=lax.Precision.HIGHEST)
    nin_ref[...] = lax.rsqrt(deg_d)[:, None]


def _tc_final_body(hs_ref, agg_ref, w_ref, b_ref, nin_ref, out_ref):
    agg = agg_ref[0] + agg_ref[1]
    h = lax.dot_general(
        agg, w_ref[...], (((1,), (1,)), ((), ())),
        preferred_element_type=jnp.float32,
        precision=lax.Precision.HIGHEST) + b_ref[...]
    out_ref[...] = hs_ref[...] + h * nin_ref[...]


RBLK = 2000


def kernel(feature, edge_index, e_w, snorm_n, snorm_e, W_self, W, b):
    ei = edge_index.astype(jnp.int32)
    src_h = ei[0].reshape(NW, CPT, CHUNK)
    dst_h = ei[1].reshape(NW, CPT, CHUNK)
    src_a = ei[0].reshape(NW, ACP, ACH)
    dst_a = ei[1].reshape(NW, APR, 2, ACH)
    ew_a = e_w[:, 0].reshape(NW, APR, 2, ACH)

    cnt_src, cnt_dst = _sc_degree(src_h, dst_h)
    cnt_src = cnt_src.reshape(NW, N)
    cnt_dst = cnt_dst.reshape(NW, N)

    feat, hs, nin = pl.pallas_call(
        _tc_prep_body,
        out_shape=[
            jax.ShapeDtypeStruct((N, D), jnp.float32),
            jax.ShapeDtypeStruct((N, D), jnp.float32),
            jax.ShapeDtypeStruct((N, 1), jnp.float32),
        ],
    )(cnt_src, cnt_dst, feature, W_self)

    aggp = _sc_aggregate(feat, src_a, dst_a, ew_a)

    h = pl.pallas_call(
        _tc_final_body,
        out_shape=jax.ShapeDtypeStruct((N, D), jnp.float32),
    )(hs, aggp, W, b.reshape(1, D), nin)

    return h, e_w


# speedup vs baseline: 1.0198x; 1.0198x over previous
import dataclasses
import functools

import jax
import jax.numpy as jnp
from jax import lax
from jax.experimental import pallas as pl
from jax.experimental.pallas import tpu as pltpu
from jax.experimental.pallas import tpu_sc as plsc

N = 10000
E = 320000
D = 128
NC = 2
NS = 16
NW = NC * NS
CHUNK = 80
EPT = E // NW
CPT = EPT // CHUNK
ACH = 125
ACP = EPT // ACH
APR = ACP // 2
RPT = N // NS

_vmesh = plsc.VectorSubcoreMesh(core_axis_name="c", subcore_axis_name="s")

_sc_cp = pltpu.CompilerParams()
if "needs_layout_passes" in pltpu.CompilerParams.__dataclass_fields__:
    _sc_cp = dataclasses.replace(_sc_cp, needs_layout_passes=False)


@functools.partial(
    pl.kernel,
    out_type=(jax.ShapeDtypeStruct((NW, 1, N), jnp.float32),
              jax.ShapeDtypeStruct((NW, 1, N), jnp.float32)),
    mesh=_vmesh,
    scratch_types=[
        pltpu.VMEM((1, N), jnp.float32),
        pltpu.VMEM((1, N), jnp.float32),
        pltpu.VMEM((CPT, CHUNK), jnp.int32),
    ],
    compiler_params=_sc_cp,
)
def _sc_degree(src_hbm, dst_hbm, osrc_hbm, odst_hbm, cs_v, cd_v, idx_v):
    c = lax.axis_index("c")
    s = lax.axis_index("s")
    w = c * NS + s
    zero16 = jnp.zeros((16,), jnp.float32)

    @pl.loop(0, N // 16)
    def _(g):
        cs_v[0, pl.ds(g * 16, 16)] = zero16
        cd_v[0, pl.ds(g * 16, 16)] = zero16

    ones = jnp.ones((16,), jnp.float32)
    zidx = jnp.zeros((16,), jnp.int32)

    pltpu.sync_copy(src_hbm.at[w], idx_v)

    @pl.loop(0, CPT)
    def _(r):
        for g in range(CHUNK // 16):
            plsc.addupdate_scatter(
                cs_v, [zidx, idx_v[r, pl.ds(g * 16, 16)]], ones)

    pltpu.sync_copy(dst_hbm.at[w], idx_v)

    @pl.loop(0, CPT)
    def _(r):
        for g in range(CHUNK // 16):
            plsc.addupdate_scatter(
                cd_v, [zidx, idx_v[r, pl.ds(g * 16, 16)]], ones)

    pltpu.sync_copy(cs_v, osrc_hbm.at[w])
    pltpu.sync_copy(cd_v, odst_hbm.at[w])


@functools.partial(
    pl.kernel,
    out_type=jax.ShapeDtypeStruct((NC, N, D), jnp.float32),
    mesh=_vmesh,
    scratch_types=[
        [pltpu.VMEM((ACH, D), jnp.float32) for _ in range(2)],
        pltpu.VMEM((ACP, ACH), jnp.int32),
        [pltpu.VMEM((2, ACH), jnp.int32) for _ in range(2)],
        [pltpu.VMEM((2, ACH), jnp.float32) for _ in range(2)],
        pltpu.VMEM_SHARED((N, D), jnp.float32),
        [pltpu.SemaphoreType.DMA for _ in range(2)],
        [pltpu.SemaphoreType.DMA for _ in range(2)],
        [pltpu.SemaphoreType.DMA for _ in range(2)],
    ],
    compiler_params=_sc_cp,
)
def _sc_aggregate(feat_hbm, src_hbm, dst_hbm, ew_hbm, out_hbm,
                  rows, src_v, dstp, ewp, acc_sh, sg, ss, sa):
    c = lax.axis_index("c")
    s = lax.axis_index("s")
    w = c * NS + s

    zero16 = jnp.zeros((16,), jnp.float32)

    @pl.loop(0, ACH)
    def _(r):
        for v in range(D // 16):
            rows[0][r, pl.ds(v * 16, 16)] = zero16

    @pl.loop(s, N // 80, step=NS)
    def _(k):
        pltpu.async_copy(rows[0].at[pl.ds(0, 80)],
                         acc_sh.at[pl.ds(k * 80, 80)], sg[0])

    @pl.loop(s, N // 80, step=NS)
    def _(k):
        pltpu.make_async_copy(rows[0].at[pl.ds(0, 80)],
                              acc_sh.at[pl.ds(k * 80, 80)], sg[0]).wait()

    plsc.subcore_barrier()

    pltpu.sync_copy(src_hbm.at[w], src_v)

    def _aux_start(p, a):
        pltpu.async_copy(dst_hbm.at[w].at[p], dstp[a], sa[a])
        pltpu.async_copy(ew_hbm.at[w].at[p], ewp[a], sa[a])

    def _aux_wait(a):
        pltpu.make_async_copy(dst_hbm.at[w].at[0], dstp[a], sa[a]).wait()
        pltpu.make_async_copy(ew_hbm.at[w].at[0], ewp[a], sa[a]).wait()

    def _gather_start(k, j):
        pltpu.async_copy(feat_hbm.at[src_v.at[k]], rows[j], sg[j])

    def _gather_wait(j):
        pltpu.make_async_copy(feat_hbm.at[src_v.at[0]], rows[j],
                              sg[j]).wait()

    def _scatter_start(j, a, i):
        pltpu.async_copy(rows[j], acc_sh.at[dstp[a].at[i]], ss[j], add=True)

    def _scatter_wait(j, a, i):
        pltpu.make_async_copy(rows[j], acc_sh.at[dstp[a].at[i]],
                              ss[j]).wait()

    def _scale(j, a, i):
        ii = jnp.broadcast_to(i, (16,)).astype(jnp.int32)

        @pl.loop(0, ACH, unroll=4)
        def _(r):
            ewb = plsc.load_gather(
                ewp[a], [ii, jnp.broadcast_to(r, (16,)).astype(jnp.int32)])
            for v in range(D // 16):
                sl = pl.ds(v * 16, 16)
                rows[j][r, sl] = rows[j][r, sl] * ewb

    _aux_start(0, 0)
    _aux_start(1, 1)
    _gather_start(0, 0)
    _gather_start(1, 1)

    @pl.loop(0, ACP // 4)
    def _(u):
        k0 = 4 * u
        p0 = 2 * u
        _aux_wait(0)
        _gather_wait(0)
        _scale(0, 0, 0)
        _scatter_start(0, 0, 0)
        _gather_wait(1)
        _scale(1, 0, 1)
        _scatter_start(1, 0, 1)
        _scatter_wait(0, 0, 0)
        _gather_start(k0 + 2, 0)
        _scatter_wait(1, 0, 1)
        _gather_start(k0 + 3, 1)
        _aux_start(jnp.minimum(p0 + 2, APR - 2), 0)
        _aux_wait(1)
        _gather_wait(0)
        _scale(0, 1, 0)
        _scatter_start(0, 1, 0)
        _gather_wait(1)
        _scale(1, 1, 1)
        _scatter_start(1, 1, 1)
        _scatter_wait(0, 1, 0)
        _gather_start(jnp.minimum(k0 + 4, ACP - 4), 0)
        _scatter_wait(1, 1, 1)
        _gather_start(jnp.minimum(k0 + 5, ACP - 3), 1)
        _aux_start(jnp.minimum(p0 + 3, APR - 1), 1)

    _gather_wait(0)
    _gather_wait(1)
    _aux_wait(0)
    _aux_wait(1)

    plsc.subcore_barrier()

    @pl.loop(s, N // 80, step=NS)
    def _(k):
        pltpu.async_copy(acc_sh.at[pl.ds(k * 80, 80)],
                         out_hbm.at[c].at[pl.ds(k * 80, 80)], sg[0])

    @pl.loop(s, N // 80, step=NS)
    def _(k):
        pltpu.make_async_copy(acc_sh.at[pl.ds(k * 80, 80)],
                              out_hbm.at[c].at[pl.ds(k * 80, 80)],
                              sg[0]).wait()


def _tc_prep_body(cs_ref, cd_ref, x_ref, wself_ref, feat_ref, hs_ref, nin_ref):
    deg_s = jnp.maximum(jnp.sum(cs_ref[...], axis=0), 1.0)
    deg_d = jnp.maximum(jnp.sum(cd_ref[...], axis=0), 1.0)
    x = x_ref[...]
    feat_ref[...] = x * lax.rsqrt(deg_s)[:, None]
    hs_ref[...] = lax.dot_general(
        x, wself_ref[...], (((1,), (1,)), ((), ())),
        preferred_element_type=jnp.float32,
        precision=lax.Precision.HIGHEST)
    nin_ref[...] = lax.rsqrt(deg_d)[:, None]


def _tc_final_body(hs_ref, agg_ref, w_ref, b_ref, nin_ref, out_ref):
    agg = agg_ref[0] + agg_ref[1]
    h = lax.dot_general(
        agg, w_ref[...], (((1,), (1,)), ((), ())),
        preferred_element_type=jnp.float32,
        precision=lax.Precision.HIGHEST) + b_ref[...]
    out_ref[...] = hs_ref[...] + h * nin_ref[...]


RBLK = 2000


def kernel(feature, edge_index, e_w, snorm_n, snorm_e, W_self, W, b):
    ei = edge_index.astype(jnp.int32)
    src_h = ei[0].reshape(NW, CPT, CHUNK)
    dst_h = ei[1].reshape(NW, CPT, CHUNK)
    src_a = ei[0].reshape(NW, ACP, ACH)
    dst_a = ei[1].reshape(NW, APR, 2, ACH)
    ew_a = e_w[:, 0].reshape(NW, APR, 2, ACH)

    cnt_src, cnt_dst = _sc_degree(src_h, dst_h)
    cnt_src = cnt_src.reshape(NW, N)
    cnt_dst = cnt_dst.reshape(NW, N)

    feat, hs, nin = pl.pallas_call(
        _tc_prep_body,
        out_shape=[
            jax.ShapeDtypeStruct((N, D), jnp.float32),
            jax.ShapeDtypeStruct((N, D), jnp.float32),
            jax.ShapeDtypeStruct((N, 1), jnp.float32),
        ],
    )(cnt_src, cnt_dst, feature, W_self)

    aggp = _sc_aggregate(feat, src_a, dst_a, ew_a)

    h = pl.pallas_call(
        _tc_final_body,
        out_shape=jax.ShapeDtypeStruct((N, D), jnp.float32),
    )(hs, aggp, W, b.reshape(1, D), nin)

    return h, e_w


# speedup vs baseline: 1.0211x; 1.0013x over previous
import dataclasses
import functools

import jax
import jax.numpy as jnp
from jax import lax
from jax.experimental import pallas as pl
from jax.experimental.pallas import tpu as pltpu
from jax.experimental.pallas import tpu_sc as plsc

N = 10000
E = 320000
D = 128
NC = 2
NS = 16
NW = NC * NS
CHUNK = 80
EPT = E // NW
CPT = EPT // CHUNK
ACH = 125
ACP = EPT // ACH
APR = ACP // 2
RPT = N // NS

_vmesh = plsc.VectorSubcoreMesh(core_axis_name="c", subcore_axis_name="s")

_sc_cp = pltpu.CompilerParams()
if "needs_layout_passes" in pltpu.CompilerParams.__dataclass_fields__:
    _sc_cp = dataclasses.replace(_sc_cp, needs_layout_passes=False)


@functools.partial(
    pl.kernel,
    out_type=(jax.ShapeDtypeStruct((NW, 1, N), jnp.float32),
              jax.ShapeDtypeStruct((NW, 1, N), jnp.float32)),
    mesh=_vmesh,
    scratch_types=[
        pltpu.VMEM((1, N), jnp.float32),
        pltpu.VMEM((1, N), jnp.float32),
        pltpu.VMEM((CPT, CHUNK), jnp.int32),
    ],
    compiler_params=_sc_cp,
)
def _sc_degree(src_hbm, dst_hbm, osrc_hbm, odst_hbm, cs_v, cd_v, idx_v):
    c = lax.axis_index("c")
    s = lax.axis_index("s")
    w = c * NS + s
    zero16 = jnp.zeros((16,), jnp.float32)

    @pl.loop(0, N // 16)
    def _(g):
        cs_v[0, pl.ds(g * 16, 16)] = zero16
        cd_v[0, pl.ds(g * 16, 16)] = zero16

    ones = jnp.ones((16,), jnp.float32)
    zidx = jnp.zeros((16,), jnp.int32)

    pltpu.sync_copy(src_hbm.at[w], idx_v)

    @pl.loop(0, CPT)
    def _(r):
        for g in range(CHUNK // 16):
            plsc.addupdate_scatter(
                cs_v, [zidx, idx_v[r, pl.ds(g * 16, 16)]], ones)

    pltpu.sync_copy(dst_hbm.at[w], idx_v)

    @pl.loop(0, CPT)
    def _(r):
        for g in range(CHUNK // 16):
            plsc.addupdate_scatter(
                cd_v, [zidx, idx_v[r, pl.ds(g * 16, 16)]], ones)

    pltpu.sync_copy(cs_v, osrc_hbm.at[w])
    pltpu.sync_copy(cd_v, odst_hbm.at[w])


@functools.partial(
    pl.kernel,
    out_type=jax.ShapeDtypeStruct((NC, N, D), jnp.float32),
    mesh=_vmesh,
    scratch_types=[
        [pltpu.VMEM((ACH, D), jnp.float32) for _ in range(2)],
        pltpu.VMEM((ACP, ACH), jnp.int32),
        [pltpu.VMEM((2, ACH), jnp.int32) for _ in range(2)],
        [pltpu.VMEM((2, ACH), jnp.float32) for _ in range(2)],
        pltpu.VMEM_SHARED((N, D), jnp.float32),
        [pltpu.SemaphoreType.DMA for _ in range(2)],
        [pltpu.SemaphoreType.DMA for _ in range(2)],
        [pltpu.SemaphoreType.DMA for _ in range(2)],
    ],
    compiler_params=_sc_cp,
)
def _sc_aggregate(feat_hbm, src_hbm, dst_hbm, ew_hbm, out_hbm,
                  rows, src_v, dstp, ewp, acc_sh, sg, ss, sa):
    c = lax.axis_index("c")
    s = lax.axis_index("s")
    w = c * NS + s

    zero16 = jnp.zeros((16,), jnp.float32)

    @pl.loop(0, ACH)
    def _(r):
        for v in range(D // 16):
            rows[0][r, pl.ds(v * 16, 16)] = zero16

    @pl.loop(s, N // 80, step=NS)
    def _(k):
        pltpu.async_copy(rows[0].at[pl.ds(0, 80)],
                         acc_sh.at[pl.ds(k * 80, 80)], sg[0])

    @pl.loop(s, N // 80, step=NS)
    def _(k):
        pltpu.make_async_copy(rows[0].at[pl.ds(0, 80)],
                              acc_sh.at[pl.ds(k * 80, 80)], sg[0]).wait()

    plsc.subcore_barrier()

    pltpu.sync_copy(src_hbm.at[w], src_v)

    def _aux_start(p, a):
        pltpu.async_copy(dst_hbm.at[w].at[p], dstp[a], sa[a])
        pltpu.async_copy(ew_hbm.at[w].at[p], ewp[a], sa[a])

    def _aux_wait(a):
        pltpu.make_async_copy(dst_hbm.at[w].at[0], dstp[a], sa[a]).wait()
        pltpu.make_async_copy(ew_hbm.at[w].at[0], ewp[a], sa[a]).wait()

    def _gather_start(k, j):
        pltpu.async_copy(feat_hbm.at[src_v.at[k]], rows[j], sg[j])

    def _gather_wait(j):
        pltpu.make_async_copy(feat_hbm.at[src_v.at[0]], rows[j],
                              sg[j]).wait()

    def _scatter_start(j, a, i):
        pltpu.async_copy(rows[j], acc_sh.at[dstp[a].at[i]], ss[j], add=True)

    def _scatter_wait(j, a, i):
        pltpu.make_async_copy(rows[j], acc_sh.at[dstp[a].at[i]],
                              ss[j]).wait()

    def _scale(j, a, i):
        ii = jnp.broadcast_to(i, (16,)).astype(jnp.int32)

        @pl.loop(0, ACH, unroll=5)
        def _(r):
            ewb = plsc.load_gather(
                ewp[a], [ii, jnp.broadcast_to(r, (16,)).astype(jnp.int32)])
            for v in range(D // 16):
                sl = pl.ds(v * 16, 16)
                rows[j][r, sl] = rows[j][r, sl] * ewb

    _aux_start(0, 0)
    _aux_start(1, 1)
    _gather_start(0, 0)
    _gather_start(1, 1)

    @pl.loop(0, ACP // 4)
    def _(u):
        k0 = 4 * u
        p0 = 2 * u
        _aux_wait(0)
        _gather_wait(0)
        _scale(0, 0, 0)
        _scatter_start(0, 0, 0)
        _gather_wait(1)
        _scale(1, 0, 1)
        _scatter_start(1, 0, 1)
        _scatter_wait(0, 0, 0)
        _gather_start(k0 + 2, 0)
        _scatter_wait(1, 0, 1)
        _gather_start(k0 + 3, 1)
        _aux_start(jnp.minimum(p0 + 2, APR - 2), 0)
        _aux_wait(1)
        _gather_wait(0)
        _scale(0, 1, 0)
        _scatter_start(0, 1, 0)
        _gather_wait(1)
        _scale(1, 1, 1)
        _scatter_start(1, 1, 1)
        _scatter_wait(0, 1, 0)
        _gather_start(jnp.minimum(k0 + 4, ACP - 4), 0)
        _scatter_wait(1, 1, 1)
        _gather_start(jnp.minimum(k0 + 5, ACP - 3), 1)
        _aux_start(jnp.minimum(p0 + 3, APR - 1), 1)

    _gather_wait(0)
    _gather_wait(1)
    _aux_wait(0)
    _aux_wait(1)

    plsc.subcore_barrier()

    @pl.loop(s, N // 80, step=NS)
    def _(k):
        pltpu.async_copy(acc_sh.at[pl.ds(k * 80, 80)],
                         out_hbm.at[c].at[pl.ds(k * 80, 80)], sg[0])

    @pl.loop(s, N // 80, step=NS)
    def _(k):
        pltpu.make_async_copy(acc_sh.at[pl.ds(k * 80, 80)],
                              out_hbm.at[c].at[pl.ds(k * 80, 80)],
                              sg[0]).wait()


def _tc_prep_body(cs_ref, cd_ref, x_ref, wself_ref, feat_ref, hs_ref, nin_ref):
    deg_s = jnp.maximum(jnp.sum(cs_ref[...], axis=0), 1.0)
    deg_d = jnp.maximum(jnp.sum(cd_ref[...], axis=0), 1.0)
    x = x_ref[...]
    feat_ref[...] = x * lax.rsqrt(deg_s)[:, None]
    hs_ref[...] = lax.dot_general(
        x, wself_ref[...], (((1,), (1,)), ((), ())),
        preferred_element_type=jnp.float32,
        precision=lax.Precision.HIGHEST)
    nin_ref[...] = lax.rsqrt(deg_d)[:, None]


def _tc_final_body(hs_ref, agg_ref, w_ref, b_ref, nin_ref, out_ref):
    agg = agg_ref[0] + agg_ref[1]
    h = lax.dot_general(
        agg, w_ref[...], (((1,), (1,)), ((), ())),
        preferred_element_type=jnp.float32,
        precision=lax.Precision.HIGHEST) + b_ref[...]
    out_ref[...] = hs_ref[...] + h * nin_ref[...]


RBLK = 2000


def kernel(feature, edge_index, e_w, snorm_n, snorm_e, W_self, W, b):
    ei = edge_index.astype(jnp.int32)
    src_h = ei[0].reshape(NW, CPT, CHUNK)
    dst_h = ei[1].reshape(NW, CPT, CHUNK)
    src_a = ei[0].reshape(NW, ACP, ACH)
    dst_a = ei[1].reshape(NW, APR, 2, ACH)
    ew_a = e_w[:, 0].reshape(NW, APR, 2, ACH)

    cnt_src, cnt_dst = _sc_degree(src_h, dst_h)
    cnt_src = cnt_src.reshape(NW, N)
    cnt_dst = cnt_dst.reshape(NW, N)

    feat, hs, nin = pl.pallas_call(
        _tc_prep_body,
        out_shape=[
            jax.ShapeDtypeStruct((N, D), jnp.float32),
            jax.ShapeDtypeStruct((N, D), jnp.float32),
            jax.ShapeDtypeStruct((N, 1), jnp.float32),
        ],
    )(cnt_src, cnt_dst, feature, W_self)

    aggp = _sc_aggregate(feat, src_a, dst_a, ew_a)

    h = pl.pallas_call(
        _tc_final_body,
        out_shape=jax.ShapeDtypeStruct((N, D), jnp.float32),
    )(hs, aggp, W, b.reshape(1, D), nin)

    return h, e_w
